# baseline (speedup 1.0000x reference)
import jax
import jax.numpy as jnp
from jax.experimental import pallas as pl

B, T, D, E = 4, 4096, 4096, 64
ROWS = 16384
BLK = 1024


def _router_kernel(x_ref, sigt_ref, idx_ref, probs_ref):
    sgn = jnp.sign(sigt_ref[...]).astype(jnp.bfloat16)
    xb = x_ref[...].astype(jnp.bfloat16)
    scores = jnp.dot(xb, sgn, preferred_element_type=jnp.float32)

    m = jnp.max(scores, axis=1, keepdims=True)
    col = jax.lax.broadcasted_iota(jnp.int32, scores.shape, 1)
    idx_ref[...] = jnp.min(jnp.where(scores == m, col, E), axis=1,
                           keepdims=True)

    e = jnp.exp(scores - m)
    probs_ref[...] = e / jnp.sum(e, axis=1, keepdims=True)


def kernel(x, signatures):
    x2 = x.reshape(ROWS, D)
    sigt = signatures.T

    grid = (ROWS // BLK,)
    idx, probs = pl.pallas_call(
        _router_kernel,
        grid=grid,
        in_specs=[
            pl.BlockSpec((BLK, D), lambda i: (i, 0)),
            pl.BlockSpec((D, E), lambda i: (0, 0)),
        ],
        out_specs=[
            pl.BlockSpec((BLK, 1), lambda i: (i, 0)),
            pl.BlockSpec((BLK, E), lambda i: (i, 0)),
        ],
        out_shape=[
            jax.ShapeDtypeStruct((ROWS, 1), jnp.int32),
            jax.ShapeDtypeStruct((ROWS, E), jnp.float32),
        ],
    )(x2, sigt)

    return idx.reshape(B, T), probs.reshape(B, T, E)
